# Initial kernel scaffold; baseline (speedup 1.0000x reference)
import jax
import jax.numpy as jnp
from jax.experimental import pallas as pl


def kernel(input_features, levels, level_deltas, weight, bias):
    raise NotImplementedError("write your pallas kernel here")


# speedup vs baseline: 1.2786x; 1.2786x over previous
import functools

import jax
import jax.numpy as jnp
from jax.experimental import pallas as pl
from jax.experimental.pallas import tpu as pltpu

P = 1048576
CIN = 32
COUT = 32
S = 4


def _body(ld_ref, lev_ref, x_ref, w_ref, b_ref, o_ref):
    delta = ld_ref[0]
    s = jnp.clip(lev_ref[:] + delta, 0, S - 1)
    xb = x_ref[0]
    y = jnp.dot(w_ref[:], xb, preferred_element_type=jnp.float32)
    acc = b_ref[:] + jnp.where(s == 0, y[0:COUT, :], 0.0)
    for k in range(1, S):
        acc = acc + jnp.where(s == k, y[k * COUT:(k + 1) * COUT, :], 0.0)
    o_ref[0] = acc


@functools.partial(jax.jit, static_argnames=("block",))
def _run(x, levels2d, level_deltas, wstack, bias2d, block=2048):
    p = x.shape[2]
    grid = (p // block,)
    return pl.pallas_call(
        _body,
        grid=grid,
        in_specs=[
            pl.BlockSpec(memory_space=pltpu.SMEM),
            pl.BlockSpec((1, block), lambda i: (0, i)),
            pl.BlockSpec((1, CIN, block), lambda i: (0, 0, i)),
            pl.BlockSpec((S * COUT, CIN), lambda i: (0, 0)),
            pl.BlockSpec((COUT, 1), lambda i: (0, 0)),
        ],
        out_specs=pl.BlockSpec((1, COUT, block), lambda i: (0, 0, i)),
        out_shape=jax.ShapeDtypeStruct((1, COUT, p), x.dtype),
        compiler_params=pltpu.CompilerParams(
            dimension_semantics=("arbitrary",),
        ),
    )(level_deltas, levels2d, x, wstack, bias2d)


def kernel(input_features, levels, level_deltas, weight, bias):
    wstack = weight.reshape(S * COUT, CIN)
    levels2d = levels.reshape(1, -1)
    bias2d = bias.reshape(COUT, 1)
    return _run(input_features, levels2d, level_deltas, wstack, bias2d)


# speedup vs baseline: 2.8666x; 2.2419x over previous
import functools

import jax
import jax.numpy as jnp
from jax.experimental import pallas as pl
from jax.experimental.pallas import tpu as pltpu

P = 1048576
CIN = 32
COUT = 32
S = 4


def _body(ld_ref, lev_ref, x_ref, w_ref, b_ref, o_ref):
    delta = ld_ref[0]
    s = jnp.clip(lev_ref[:] + delta, 0, S - 1)
    xb = x_ref[0]
    y = jnp.dot(w_ref[:], xb, preferred_element_type=jnp.float32)
    acc = b_ref[:] + jnp.where(s == 0, y[0:COUT, :], 0.0)
    for k in range(1, S):
        acc = acc + jnp.where(s == k, y[k * COUT:(k + 1) * COUT, :], 0.0)
    o_ref[0] = acc


@functools.partial(jax.jit, static_argnames=("block",))
def _run(x, levels2d, level_deltas, wstack, bias2d, block=2048):
    p = x.shape[2]
    grid = (p // block,)
    return pl.pallas_call(
        _body,
        grid=grid,
        in_specs=[
            pl.BlockSpec(memory_space=pltpu.SMEM),
            pl.BlockSpec((1, block), lambda i: (0, i)),
            pl.BlockSpec((1, CIN, block), lambda i: (0, 0, i)),
            pl.BlockSpec((S * COUT, CIN), lambda i: (0, 0)),
            pl.BlockSpec((COUT, 1), lambda i: (0, 0)),
        ],
        out_specs=pl.BlockSpec((1, COUT, block), lambda i: (0, 0, i)),
        out_shape=jax.ShapeDtypeStruct((1, COUT, p), x.dtype),
        compiler_params=pltpu.CompilerParams(
            dimension_semantics=("parallel",),
        ),
    )(level_deltas, levels2d, x, wstack, bias2d)


def kernel(input_features, levels, level_deltas, weight, bias):
    wstack = weight.reshape(S * COUT, CIN)
    levels2d = levels.reshape(1, -1)
    bias2d = bias.reshape(COUT, 1)
    return _run(input_features, levels2d, level_deltas, wstack, bias2d,
                block=8192)


# speedup vs baseline: 3.7453x; 1.3065x over previous
import functools

import jax
import jax.numpy as jnp
from jax.experimental import pallas as pl
from jax.experimental.pallas import tpu as pltpu

P = 1048576
CIN = 32
COUT = 32
S = 4


def _body(ld_ref, lev_ref, x_ref, w_ref, b_ref, o_ref):
    delta = ld_ref[0]
    s = jnp.clip(lev_ref[:] + delta, 0, S - 1)
    xb = x_ref[0]
    y = jnp.dot(w_ref[:], xb, preferred_element_type=jnp.float32)
    acc = b_ref[:] + jnp.where(s == 0, y[0:COUT, :], 0.0)
    for k in range(1, S):
        acc = acc + jnp.where(s == k, y[k * COUT:(k + 1) * COUT, :], 0.0)
    o_ref[0] = acc


@functools.partial(jax.jit, static_argnames=("block",))
def _run(x, levels2d, level_deltas, wstack, bias2d, block=2048):
    p = x.shape[2]
    grid = (p // block,)
    return pl.pallas_call(
        _body,
        grid=grid,
        in_specs=[
            pl.BlockSpec(memory_space=pltpu.SMEM),
            pl.BlockSpec((1, block), lambda i: (0, i)),
            pl.BlockSpec((1, CIN, block), lambda i: (0, 0, i)),
            pl.BlockSpec((S * COUT, CIN), lambda i: (0, 0)),
            pl.BlockSpec((COUT, 1), lambda i: (0, 0)),
        ],
        out_specs=pl.BlockSpec((1, COUT, block), lambda i: (0, 0, i)),
        out_shape=jax.ShapeDtypeStruct((1, COUT, p), x.dtype),
        compiler_params=pltpu.CompilerParams(
            dimension_semantics=("parallel",),
        ),
    )(level_deltas, levels2d, x, wstack, bias2d)


def kernel(input_features, levels, level_deltas, weight, bias):
    wstack = weight.reshape(S * COUT, CIN)
    levels2d = levels.reshape(1, -1)
    bias2d = bias.reshape(COUT, 1)
    return _run(input_features, levels2d, level_deltas, wstack, bias2d,
                block=16384)


# speedup vs baseline: 4.4313x; 1.1832x over previous
import functools

import jax
import jax.numpy as jnp
from jax.experimental import pallas as pl
from jax.experimental.pallas import tpu as pltpu

P = 1048576
CIN = 32
COUT = 32
S = 4


def _body(ld_ref, lev_ref, x_ref, w_ref, b_ref, o_ref):
    delta = ld_ref[0]
    s = jnp.clip(lev_ref[:] + delta, 0, S - 1)
    xb = x_ref[0]
    y = jnp.dot(w_ref[:], xb, preferred_element_type=jnp.float32)
    acc = b_ref[:] + jnp.where(s == 0, y[0:COUT, :], 0.0)
    for k in range(1, S):
        acc = acc + jnp.where(s == k, y[k * COUT:(k + 1) * COUT, :], 0.0)
    o_ref[0] = acc


@functools.partial(jax.jit, static_argnames=("block",))
def _run(x, levels2d, level_deltas, wstack, bias2d, block=2048):
    p = x.shape[2]
    grid = (p // block,)
    return pl.pallas_call(
        _body,
        grid=grid,
        in_specs=[
            pl.BlockSpec(memory_space=pltpu.SMEM),
            pl.BlockSpec((1, block), lambda i: (0, i)),
            pl.BlockSpec((1, CIN, block), lambda i: (0, 0, i)),
            pl.BlockSpec((S * COUT, CIN), lambda i: (0, 0)),
            pl.BlockSpec((COUT, 1), lambda i: (0, 0)),
        ],
        out_specs=pl.BlockSpec((1, COUT, block), lambda i: (0, 0, i)),
        out_shape=jax.ShapeDtypeStruct((1, COUT, p), x.dtype),
        compiler_params=pltpu.CompilerParams(
            dimension_semantics=("parallel",),
        ),
    )(level_deltas, levels2d, x, wstack, bias2d)


def kernel(input_features, levels, level_deltas, weight, bias):
    wstack = weight.reshape(S * COUT, CIN)
    levels2d = levels.reshape(1, -1)
    bias2d = bias.reshape(COUT, 1)
    return _run(input_features, levels2d, level_deltas, wstack, bias2d,
                block=32768)


# speedup vs baseline: 4.7920x; 1.0814x over previous
import functools

import jax
import jax.numpy as jnp
from jax.experimental import pallas as pl
from jax.experimental.pallas import tpu as pltpu

P = 1048576
CIN = 32
COUT = 32
S = 4


def _body(ld_ref, lev_ref, x_ref, w_ref, b_ref, o_ref):
    delta = ld_ref[0]
    s = jnp.clip(lev_ref[:] + delta, 0, S - 1)
    xb = x_ref[0]
    y = jnp.dot(w_ref[:], xb, preferred_element_type=jnp.float32)
    acc = b_ref[:] + jnp.where(s == 0, y[0:COUT, :], 0.0)
    for k in range(1, S):
        acc = acc + jnp.where(s == k, y[k * COUT:(k + 1) * COUT, :], 0.0)
    o_ref[0] = acc


@functools.partial(jax.jit, static_argnames=("block",))
def _run(x, levels2d, level_deltas, wstack, bias2d, block=2048):
    p = x.shape[2]
    grid = (p // block,)
    return pl.pallas_call(
        _body,
        grid=grid,
        in_specs=[
            pl.BlockSpec(memory_space=pltpu.SMEM),
            pl.BlockSpec((1, block), lambda i: (0, i)),
            pl.BlockSpec((1, CIN, block), lambda i: (0, 0, i)),
            pl.BlockSpec((S * COUT, CIN), lambda i: (0, 0)),
            pl.BlockSpec((COUT, 1), lambda i: (0, 0)),
        ],
        out_specs=pl.BlockSpec((1, COUT, block), lambda i: (0, 0, i)),
        out_shape=jax.ShapeDtypeStruct((1, COUT, p), x.dtype),
        compiler_params=pltpu.CompilerParams(
            dimension_semantics=("parallel",),
        ),
    )(level_deltas, levels2d, x, wstack, bias2d)


def kernel(input_features, levels, level_deltas, weight, bias):
    wstack = weight.reshape(S * COUT, CIN)
    levels2d = levels.reshape(1, -1)
    bias2d = bias.reshape(COUT, 1)
    return _run(input_features, levels2d, level_deltas, wstack, bias2d,
                block=65536)
